# baseline (speedup 1.0000x reference)
import functools

import jax
import jax.numpy as jnp
from jax import lax
from jax.experimental import pallas as pl
from jax.experimental.pallas import tpu as pltpu
from jax.experimental.pallas import tpu_sc as plsc

BATCH = 16384
D = 32
NW = 32
ROWS_PER_W = BATCH // NW
CHUNK = 128
NCHUNK = ROWS_PER_W // CHUNK


def _sc_gather_body(uidx_hbm, iidx_hbm, umf_hbm, imf_hbm, umlp_hbm, imlp_hbm,
                    out_umf, out_imf, out_umlp, out_imlp,
                    uidx_v, iidx_v, r_umf, r_imf, r_umlp, r_imlp, sem):
    c = lax.axis_index("c")
    s = lax.axis_index("s")
    wid = s * 2 + c
    rbase = wid * NCHUNK
    pltpu.sync_copy(uidx_hbm.at[pl.ds(rbase, NCHUNK)], uidx_v)
    pltpu.sync_copy(iidx_hbm.at[pl.ds(rbase, NCHUNK)], iidx_v)
    copies = []
    for j in range(NCHUNK):
        dst = pl.ds(j * CHUNK, CHUNK)
        copies.append(pltpu.async_copy(umf_hbm.at[uidx_v.at[j]], r_umf.at[dst], sem))
        copies.append(pltpu.async_copy(imf_hbm.at[iidx_v.at[j]], r_imf.at[dst], sem))
        copies.append(pltpu.async_copy(umlp_hbm.at[uidx_v.at[j]], r_umlp.at[dst], sem))
        copies.append(pltpu.async_copy(imlp_hbm.at[iidx_v.at[j]], r_imlp.at[dst], sem))
    for cp in copies:
        cp.wait()
    obase = wid * ROWS_PER_W
    pltpu.sync_copy(r_umf, out_umf.at[pl.ds(obase, ROWS_PER_W)])
    pltpu.sync_copy(r_imf, out_imf.at[pl.ds(obase, ROWS_PER_W)])
    pltpu.sync_copy(r_umlp, out_umlp.at[pl.ds(obase, ROWS_PER_W)])
    pltpu.sync_copy(r_imlp, out_imlp.at[pl.ds(obase, ROWS_PER_W)])


def _sc_gather(uidx2d, iidx2d, U_mf, I_mf, U_mlp, I_mlp):
    rows = jax.ShapeDtypeStruct((BATCH, D), jnp.float32)
    mesh = plsc.VectorSubcoreMesh(core_axis_name="c", subcore_axis_name="s")
    fn = pl.kernel(
        _sc_gather_body,
        out_type=(rows, rows, rows, rows),
        mesh=mesh,
        scratch_types=[
            pltpu.VMEM((NCHUNK, CHUNK), jnp.int32),
            pltpu.VMEM((NCHUNK, CHUNK), jnp.int32),
            pltpu.VMEM((ROWS_PER_W, D), jnp.float32),
            pltpu.VMEM((ROWS_PER_W, D), jnp.float32),
            pltpu.VMEM((ROWS_PER_W, D), jnp.float32),
            pltpu.VMEM((ROWS_PER_W, D), jnp.float32),
            pltpu.SemaphoreType.DMA,
        ],
        compiler_params=pltpu.CompilerParams(use_tc_tiling_on_sc=False),
    )
    return fn(uidx2d, iidx2d, U_mf, I_mf, U_mlp, I_mlp)


def _tc_dense_body(umlp_ref, imlp_ref, umf_ref, imf_ref,
                   w1u_ref, w1i_ref, b1_ref, w2_ref, b2_ref,
                   wl1_ref, wl2_ref, bl_ref, out_ref):
    h1 = jnp.dot(umlp_ref[...], w1u_ref[...], preferred_element_type=jnp.float32)
    h1 = h1 + jnp.dot(imlp_ref[...], w1i_ref[...], preferred_element_type=jnp.float32)
    h1 = jnp.maximum(h1 + b1_ref[...], 0.0)
    h2 = jnp.dot(h1, w2_ref[...], preferred_element_type=jnp.float32)
    h2 = jnp.maximum(h2 + b2_ref[...], 0.0)
    mf = umf_ref[...] * imf_ref[...]
    logit = (jnp.sum(h2 * wl1_ref[...], axis=1, keepdims=True)
             + jnp.sum(mf * wl2_ref[...], axis=1, keepdims=True)
             + bl_ref[0, 0])
    scaled = 2.5 * (jnp.tanh(logit) + 1.0)
    acc = jnp.zeros_like(scaled)
    for n in range(5):
        acc = acc + jax.nn.sigmoid(10.0 * (scaled - (0.5 + n)))
    out_ref[...] = acc


def _tc_dense(umlp, imlp, umf, imf, W1u, W1i, b1, W2, b2, wl1, wl2, bl, *,
              bm=2048, interpret=False):
    nblk = BATCH // bm
    row_spec = pl.BlockSpec((bm, D), lambda i: (i, 0))

    def wspec(shape):
        return pl.BlockSpec(shape, lambda i: (0, 0))

    return pl.pallas_call(
        _tc_dense_body,
        grid=(nblk,),
        in_specs=[
            row_spec, row_spec, row_spec, row_spec,
            wspec((D, D)), wspec((D, D)), wspec((1, D)),
            wspec((D, 16)), wspec((1, 16)),
            wspec((1, 16)), wspec((1, D)), wspec((1, 1)),
        ],
        out_specs=pl.BlockSpec((bm, 1), lambda i: (i, 0)),
        out_shape=jax.ShapeDtypeStruct((BATCH, 1), jnp.float32),
        compiler_params=pltpu.CompilerParams(
            dimension_semantics=("parallel",)),
        interpret=interpret,
    )(umlp, imlp, umf, imf, W1u, W1i, b1, W2, b2, wl1, wl2, bl)


def kernel(user_indices, item_indices, U_mf, I_mf, U_mlp, I_mlp,
           W1, b1, W2, b2, Wl, bl):
    uidx2d = user_indices.astype(jnp.int32).reshape(CHUNK, CHUNK)
    iidx2d = item_indices.astype(jnp.int32).reshape(CHUNK, CHUNK)
    r_umf, r_imf, r_umlp, r_imlp = _sc_gather(uidx2d, iidx2d,
                                              U_mf, I_mf, U_mlp, I_mlp)
    W1u = W1[:D]
    W1i = W1[D:]
    b1r = b1.reshape(1, D)
    b2r = b2.reshape(1, 16)
    wl1 = Wl[:16, 0].reshape(1, 16)
    wl2 = Wl[16:, 0].reshape(1, D)
    blr = bl.reshape(1, 1)
    return _tc_dense(r_umlp, r_imlp, r_umf, r_imf,
                     W1u, W1i, b1r, W2, b2r, wl1, wl2, blr)


# speedup vs baseline: 3.1618x; 3.1618x over previous
import jax
import jax.numpy as jnp
from jax import lax
from jax.experimental import pallas as pl
from jax.experimental.pallas import tpu as pltpu
from jax.experimental.pallas import tpu_sc as plsc

BATCH = 16384
D = 32
NW = 32
RPW = BATCH // NW


def _sc_gather_body(uidx_hbm, iidx_hbm, tumf, timf, tumlp, timlp,
                    out_umf, out_imf, out_umlp, out_imlp,
                    uidx_v, iidx_v, stage, obufs, sem0, sem1):
    c = lax.axis_index("c")
    s = lax.axis_index("s")
    wid = s * 2 + c
    wbase = wid * RPW
    pltpu.sync_copy(uidx_hbm.at[pl.ds(wbase, RPW)], uidx_v)
    pltpu.sync_copy(iidx_hbm.at[pl.ds(wbase, RPW)], iidx_v)
    tables = (tumf, timf, tumlp, timlp)
    sems = (sem0, sem1)
    iota = lax.iota(jnp.int32, 16)
    iota16 = iota + 16

    def splat_at(idx_v, i):
        return plsc.load_gather(idx_v, [jnp.full((16,), i, jnp.int32)])

    def issue(i, q):
        ru = jnp.max(splat_at(uidx_v, i))
        ri = jnp.max(splat_at(iidx_v, i))
        su = pl.multiple_of(ru & -128, 128)
        si = pl.multiple_of(ri & -128, 128)
        for t, tab in enumerate(tables):
            st = su if t in (0, 2) else si
            pltpu.async_copy(tab.at[:, pl.ds(st, 128)], stage.at[q, t],
                             sems[q])

    def drain(q):
        for t in range(4):
            pltpu.make_async_copy(tumf.at[:, pl.ds(0, 128)],
                                  stage.at[q, t], sems[q]).wait()

    def extract(i, q):
        cu = splat_at(uidx_v, i) & 127
        ci = splat_at(iidx_v, i) & 127
        col = jnp.full((16,), i, jnp.int32)
        for t in range(4):
            cc = cu if t in (0, 2) else ci
            v0 = plsc.load_gather(stage.at[q, t], [iota, cc])
            v1 = plsc.load_gather(stage.at[q, t], [iota16, cc])
            plsc.store_scatter(obufs.at[t], [iota, col], v0)
            plsc.store_scatter(obufs.at[t], [iota16, col], v1)

    def loop_body(ss, carry):
        for q in range(2):
            i = ss * 2 + q

            @pl.when(i >= 2)
            def _():
                drain(q)
                extract(i - 2, q)

            @pl.when(i < RPW)
            def _():
                issue(i, q)
        return carry

    lax.fori_loop(0, RPW // 2 + 1, loop_body, 0)
    outs = (out_umf, out_imf, out_umlp, out_imlp)
    for t, out in enumerate(outs):
        pltpu.sync_copy(obufs.at[t], out.at[:, pl.ds(wbase, RPW)])


def _sc_gather(uidx, iidx, tumf, timf, tumlp, timlp):
    out = jax.ShapeDtypeStruct((D, BATCH), jnp.float32)
    mesh = plsc.VectorSubcoreMesh(core_axis_name="c", subcore_axis_name="s")
    fn = pl.kernel(
        _sc_gather_body,
        out_type=(out, out, out, out),
        mesh=mesh,
        scratch_types=[
            pltpu.VMEM((RPW,), jnp.int32),
            pltpu.VMEM((RPW,), jnp.int32),
            pltpu.VMEM((2, 4, D, 128), jnp.float32),
            pltpu.VMEM((4, D, RPW), jnp.float32),
            pltpu.SemaphoreType.DMA,
            pltpu.SemaphoreType.DMA,
        ],
        compiler_params=pltpu.CompilerParams(needs_layout_passes=False),
    )
    return fn(uidx, iidx, tumf, timf, tumlp, timlp)


def _tc_dense_body(umlp_ref, imlp_ref, umf_ref, imf_ref,
                   w1u_ref, w1i_ref, b1_ref, w2_ref, b2_ref,
                   wl1_ref, wl2_ref, bl_ref, out_ref):
    h1 = jnp.dot(w1u_ref[...], umlp_ref[...],
                 preferred_element_type=jnp.float32)
    h1 = h1 + jnp.dot(w1i_ref[...], imlp_ref[...],
                      preferred_element_type=jnp.float32)
    h1 = jnp.maximum(h1 + b1_ref[...], 0.0)
    h2 = jnp.dot(w2_ref[...], h1, preferred_element_type=jnp.float32)
    h2 = jnp.maximum(h2 + b2_ref[...], 0.0)
    mf = umf_ref[...] * imf_ref[...]
    logit = (jnp.sum(h2 * wl1_ref[...], axis=0, keepdims=True)
             + jnp.sum(mf * wl2_ref[...], axis=0, keepdims=True)
             + bl_ref[0, 0])
    scaled = 2.5 * (jnp.tanh(logit) + 1.0)
    acc = jnp.zeros_like(scaled)
    for n in range(5):
        acc = acc + jax.nn.sigmoid(10.0 * (scaled - (0.5 + n)))
    out_ref[...] = acc


def _tc_dense(umlp, imlp, umf, imf,
              W1uT, W1iT, b1c, W2T, b2c, wl1c, wl2c, bl, *,
              bm=4096, interpret=False):
    nblk = BATCH // bm
    row_spec = pl.BlockSpec((D, bm), lambda i: (0, i))

    def wspec(shape):
        return pl.BlockSpec(shape, lambda i: (0, 0))

    return pl.pallas_call(
        _tc_dense_body,
        grid=(nblk,),
        in_specs=[
            row_spec, row_spec, row_spec, row_spec,
            wspec((D, D)), wspec((D, D)), wspec((D, 1)),
            wspec((16, D)), wspec((16, 1)),
            wspec((16, 1)), wspec((D, 1)), wspec((1, 1)),
        ],
        out_specs=pl.BlockSpec((1, bm), lambda i: (0, i)),
        out_shape=jax.ShapeDtypeStruct((1, BATCH), jnp.float32),
        compiler_params=pltpu.CompilerParams(
            dimension_semantics=("parallel",)),
        interpret=interpret,
    )(umlp, imlp, umf, imf, W1uT, W1iT, b1c, W2T, b2c, wl1c, wl2c, bl)


def kernel(user_indices, item_indices, U_mf, I_mf, U_mlp, I_mlp,
           W1, b1, W2, b2, Wl, bl):
    uidx = user_indices.astype(jnp.int32)
    iidx = item_indices.astype(jnp.int32)
    g_umf, g_imf, g_umlp, g_imlp = _sc_gather(
        uidx, iidx, U_mf.T, I_mf.T, U_mlp.T, I_mlp.T)
    W1uT = W1[:D].T
    W1iT = W1[D:].T
    b1c = b1.reshape(D, 1)
    b2c = b2.reshape(16, 1)
    wl1c = Wl[:16, 0].reshape(16, 1)
    wl2c = Wl[16:, 0].reshape(D, 1)
    blr = bl.reshape(1, 1)
    out = _tc_dense(g_umlp, g_imlp, g_umf, g_imf,
                    W1uT, W1iT, b1c, W2.T, b2c, wl1c, wl2c, blr)
    return out.reshape(BATCH, 1)


# speedup vs baseline: 3.9262x; 1.2418x over previous
import jax
import jax.numpy as jnp
from jax import lax
from jax.experimental import pallas as pl
from jax.experimental.pallas import tpu as pltpu
from jax.experimental.pallas import tpu_sc as plsc

BATCH = 16384
D = 32
NW = 32
RPW = BATCH // NW


DEPTH = 4
OCOLS = 256


def _sc_gather_body(uidx_hbm, iidx_hbm, tumf, timf, tumlp, timlp,
                    out_umf, out_imf, out_umlp, out_imlp,
                    uidx_v, iidx_v, stage, obufs, sem0, sem1, sem2, sem3):
    c = lax.axis_index("c")
    s = lax.axis_index("s")
    wid = s * 2 + c
    wbase = wid * RPW
    pltpu.sync_copy(uidx_hbm.at[pl.ds(wbase, RPW)], uidx_v)
    pltpu.sync_copy(iidx_hbm.at[pl.ds(wbase, RPW)], iidx_v)
    tables = (tumf, timf, tumlp, timlp)
    outs = (out_umf, out_imf, out_umlp, out_imlp)
    sems = (sem0, sem1, sem2, sem3)
    iota = lax.iota(jnp.int32, 16)
    iota16 = iota + 16

    def splat_at(idx_v, i):
        return plsc.load_gather(idx_v, [jnp.full((16,), i, jnp.int32)])

    def issue(i, q):
        ru = jnp.max(splat_at(uidx_v, i))
        ri = jnp.max(splat_at(iidx_v, i))
        su = pl.multiple_of(ru & -128, 128)
        si = pl.multiple_of(ri & -128, 128)
        for t, tab in enumerate(tables):
            st = su if t in (0, 2) else si
            pltpu.async_copy(tab.at[:, pl.ds(st, 128)], stage.at[q, t],
                             sems[q])

    def drain(q):
        for t in range(4):
            pltpu.make_async_copy(tumf.at[:, pl.ds(0, 128)],
                                  stage.at[q, t], sems[q]).wait()

    def extract(i, q):
        cu = splat_at(uidx_v, i) & 127
        ci = splat_at(iidx_v, i) & 127
        col = jnp.full((16,), i & (OCOLS - 1), jnp.int32)
        for t in range(4):
            cc = cu if t in (0, 2) else ci
            v0 = plsc.load_gather(stage.at[q, t], [iota, cc])
            v1 = plsc.load_gather(stage.at[q, t], [iota16, cc])
            plsc.store_scatter(obufs.at[t], [iota, col], v0)
            plsc.store_scatter(obufs.at[t], [iota16, col], v1)

    def loop_body(ss, carry):
        for q in range(DEPTH):
            i = ss * DEPTH + q

            @pl.when(i >= DEPTH)
            def _():
                drain(q)
                extract(i - DEPTH, q)

            @pl.when(i < RPW)
            def _():
                issue(i, q)

            @pl.when(i == OCOLS - 1 + DEPTH)
            def _():
                for t in range(4):
                    pltpu.sync_copy(obufs.at[t],
                                    outs[t].at[:, pl.ds(wbase, OCOLS)])
        return carry

    lax.fori_loop(0, RPW // DEPTH + 1, loop_body, 0)
    for t in range(4):
        pltpu.sync_copy(obufs.at[t],
                        outs[t].at[:, pl.ds(wbase + OCOLS, OCOLS)])


def _sc_gather(uidx, iidx, tumf, timf, tumlp, timlp):
    out = jax.ShapeDtypeStruct((D, BATCH), jnp.float32)
    mesh = plsc.VectorSubcoreMesh(core_axis_name="c", subcore_axis_name="s")
    fn = pl.kernel(
        _sc_gather_body,
        out_type=(out, out, out, out),
        mesh=mesh,
        scratch_types=[
            pltpu.VMEM((RPW,), jnp.int32),
            pltpu.VMEM((RPW,), jnp.int32),
            pltpu.VMEM((DEPTH, 4, D, 128), jnp.float32),
            pltpu.VMEM((4, D, OCOLS), jnp.float32),
            pltpu.SemaphoreType.DMA,
            pltpu.SemaphoreType.DMA,
            pltpu.SemaphoreType.DMA,
            pltpu.SemaphoreType.DMA,
        ],
        compiler_params=pltpu.CompilerParams(needs_layout_passes=False),
    )
    return fn(uidx, iidx, tumf, timf, tumlp, timlp)


def _tc_dense_body(umlp_ref, imlp_ref, umf_ref, imf_ref,
                   w1u_ref, w1i_ref, b1_ref, w2_ref, b2_ref,
                   wl1_ref, wl2_ref, bl_ref, out_ref):
    h1 = jnp.dot(w1u_ref[...], umlp_ref[...],
                 preferred_element_type=jnp.float32)
    h1 = h1 + jnp.dot(w1i_ref[...], imlp_ref[...],
                      preferred_element_type=jnp.float32)
    h1 = jnp.maximum(h1 + b1_ref[...], 0.0)
    h2 = jnp.dot(w2_ref[...], h1, preferred_element_type=jnp.float32)
    h2 = jnp.maximum(h2 + b2_ref[...], 0.0)
    mf = umf_ref[...] * imf_ref[...]
    logit = (jnp.sum(h2 * wl1_ref[...], axis=0, keepdims=True)
             + jnp.sum(mf * wl2_ref[...], axis=0, keepdims=True)
             + bl_ref[0, 0])
    scaled = 2.5 * (jnp.tanh(logit) + 1.0)
    acc = jnp.zeros_like(scaled)
    for n in range(5):
        acc = acc + jax.nn.sigmoid(10.0 * (scaled - (0.5 + n)))
    out_ref[...] = acc


def _tc_dense(umlp, imlp, umf, imf,
              W1uT, W1iT, b1c, W2T, b2c, wl1c, wl2c, bl, *,
              bm=4096, interpret=False):
    nblk = BATCH // bm
    row_spec = pl.BlockSpec((D, bm), lambda i: (0, i))

    def wspec(shape):
        return pl.BlockSpec(shape, lambda i: (0, 0))

    return pl.pallas_call(
        _tc_dense_body,
        grid=(nblk,),
        in_specs=[
            row_spec, row_spec, row_spec, row_spec,
            wspec((D, D)), wspec((D, D)), wspec((D, 1)),
            wspec((16, D)), wspec((16, 1)),
            wspec((16, 1)), wspec((D, 1)), wspec((1, 1)),
        ],
        out_specs=pl.BlockSpec((1, bm), lambda i: (0, i)),
        out_shape=jax.ShapeDtypeStruct((1, BATCH), jnp.float32),
        compiler_params=pltpu.CompilerParams(
            dimension_semantics=("parallel",)),
        interpret=interpret,
    )(umlp, imlp, umf, imf, W1uT, W1iT, b1c, W2T, b2c, wl1c, wl2c, bl)


def kernel(user_indices, item_indices, U_mf, I_mf, U_mlp, I_mlp,
           W1, b1, W2, b2, Wl, bl):
    uidx = user_indices.astype(jnp.int32)
    iidx = item_indices.astype(jnp.int32)
    g_umf, g_imf, g_umlp, g_imlp = _sc_gather(
        uidx, iidx, U_mf.T, I_mf.T, U_mlp.T, I_mlp.T)
    W1uT = W1[:D].T
    W1iT = W1[D:].T
    b1c = b1.reshape(D, 1)
    b2c = b2.reshape(16, 1)
    wl1c = Wl[:16, 0].reshape(16, 1)
    wl2c = Wl[16:, 0].reshape(D, 1)
    blr = bl.reshape(1, 1)
    out = _tc_dense(g_umlp, g_imlp, g_umf, g_imf,
                    W1uT, W1iT, b1c, W2.T, b2c, wl1c, wl2c, blr)
    return out.reshape(BATCH, 1)


# speedup vs baseline: 4.3633x; 1.1113x over previous
import jax
import jax.numpy as jnp
from jax import lax
from jax.experimental import pallas as pl
from jax.experimental.pallas import tpu as pltpu
from jax.experimental.pallas import tpu_sc as plsc

BATCH = 16384
D = 32
NW = 32
RPW = BATCH // NW


DEPTH = 6
OCOLS = 128


def _sc_gather_body(uidx_hbm, iidx_hbm, tumf, timf, tumlp, timlp,
                    out_umf, out_imf, out_umlp, out_imlp,
                    uidx_v, iidx_v, stage, obufs,
                    sem0, sem1, sem2, sem3, sem4, sem5):
    c = lax.axis_index("c")
    s = lax.axis_index("s")
    wid = s * 2 + c
    wbase = wid * RPW
    pltpu.sync_copy(uidx_hbm.at[pl.ds(wbase, RPW)], uidx_v)
    pltpu.sync_copy(iidx_hbm.at[pl.ds(wbase, RPW)], iidx_v)
    tables = (tumf, timf, tumlp, timlp)
    outs = (out_umf, out_imf, out_umlp, out_imlp)
    sems = (sem0, sem1, sem2, sem3, sem4, sem5)
    iota = lax.iota(jnp.int32, 16)
    iota16 = iota + 16

    def splat_at(idx_v, i):
        return plsc.load_gather(idx_v, [jnp.full((16,), i, jnp.int32)])

    def issue(i, q):
        ru = jnp.max(splat_at(uidx_v, i))
        ri = jnp.max(splat_at(iidx_v, i))
        su = pl.multiple_of(ru & -128, 128)
        si = pl.multiple_of(ri & -128, 128)
        for t, tab in enumerate(tables):
            st = su if t in (0, 2) else si
            pltpu.async_copy(tab.at[:, pl.ds(st, 128)], stage.at[q, t],
                             sems[q])

    def drain(q):
        for t in range(4):
            pltpu.make_async_copy(tumf.at[:, pl.ds(0, 128)],
                                  stage.at[q, t], sems[q]).wait()

    def extract(i, q):
        cu = splat_at(uidx_v, i) & 127
        ci = splat_at(iidx_v, i) & 127
        col = jnp.full((16,), i & (OCOLS - 1), jnp.int32)
        for t in range(4):
            cc = cu if t in (0, 2) else ci
            v0 = plsc.load_gather(stage.at[q, t], [iota, cc])
            v1 = plsc.load_gather(stage.at[q, t], [iota16, cc])
            plsc.store_scatter(obufs.at[t], [iota, col], v0)
            plsc.store_scatter(obufs.at[t], [iota16, col], v1)

    def loop_body(ss, carry):
        for q in range(DEPTH):
            i = ss * DEPTH + q
            pos = i - DEPTH

            @pl.when((i >= DEPTH) & (pos < RPW))
            def _():
                drain(q)
                extract(pos, q)

            @pl.when(i < RPW)
            def _():
                issue(i, q)

            @pl.when((pos >= 0) & (pos < RPW) & ((pos & (OCOLS - 1)) == OCOLS - 1))
            def _():
                blk = pl.multiple_of(wbase + (pos - (OCOLS - 1)), OCOLS)
                for t in range(4):
                    pltpu.sync_copy(obufs.at[t],
                                    outs[t].at[:, pl.ds(blk, OCOLS)])
        return carry

    lax.fori_loop(0, (RPW + DEPTH + DEPTH - 1) // DEPTH, loop_body, 0)


def _sc_gather(uidx, iidx, tumf, timf, tumlp, timlp):
    out = jax.ShapeDtypeStruct((D, BATCH), jnp.float32)
    mesh = plsc.VectorSubcoreMesh(core_axis_name="c", subcore_axis_name="s")
    fn = pl.kernel(
        _sc_gather_body,
        out_type=(out, out, out, out),
        mesh=mesh,
        scratch_types=[
            pltpu.VMEM((RPW,), jnp.int32),
            pltpu.VMEM((RPW,), jnp.int32),
            pltpu.VMEM((DEPTH, 4, D, 128), jnp.float32),
            pltpu.VMEM((4, D, OCOLS), jnp.float32),
            pltpu.SemaphoreType.DMA,
            pltpu.SemaphoreType.DMA,
            pltpu.SemaphoreType.DMA,
            pltpu.SemaphoreType.DMA,
            pltpu.SemaphoreType.DMA,
            pltpu.SemaphoreType.DMA,
        ],
        compiler_params=pltpu.CompilerParams(needs_layout_passes=False),
    )
    return fn(uidx, iidx, tumf, timf, tumlp, timlp)


def _tc_dense_body(umlp_ref, imlp_ref, umf_ref, imf_ref,
                   w1u_ref, w1i_ref, b1_ref, w2_ref, b2_ref,
                   wl1_ref, wl2_ref, bl_ref, out_ref):
    h1 = jnp.dot(w1u_ref[...], umlp_ref[...],
                 preferred_element_type=jnp.float32)
    h1 = h1 + jnp.dot(w1i_ref[...], imlp_ref[...],
                      preferred_element_type=jnp.float32)
    h1 = jnp.maximum(h1 + b1_ref[...], 0.0)
    h2 = jnp.dot(w2_ref[...], h1, preferred_element_type=jnp.float32)
    h2 = jnp.maximum(h2 + b2_ref[...], 0.0)
    mf = umf_ref[...] * imf_ref[...]
    logit = (jnp.sum(h2 * wl1_ref[...], axis=0, keepdims=True)
             + jnp.sum(mf * wl2_ref[...], axis=0, keepdims=True)
             + bl_ref[0, 0])
    scaled = 2.5 * (jnp.tanh(logit) + 1.0)
    acc = jnp.zeros_like(scaled)
    for n in range(5):
        acc = acc + jax.nn.sigmoid(10.0 * (scaled - (0.5 + n)))
    out_ref[...] = acc


def _tc_dense(umlp, imlp, umf, imf,
              W1uT, W1iT, b1c, W2T, b2c, wl1c, wl2c, bl, *,
              bm=4096, interpret=False):
    nblk = BATCH // bm
    row_spec = pl.BlockSpec((D, bm), lambda i: (0, i))

    def wspec(shape):
        return pl.BlockSpec(shape, lambda i: (0, 0))

    return pl.pallas_call(
        _tc_dense_body,
        grid=(nblk,),
        in_specs=[
            row_spec, row_spec, row_spec, row_spec,
            wspec((D, D)), wspec((D, D)), wspec((D, 1)),
            wspec((16, D)), wspec((16, 1)),
            wspec((16, 1)), wspec((D, 1)), wspec((1, 1)),
        ],
        out_specs=pl.BlockSpec((1, bm), lambda i: (0, i)),
        out_shape=jax.ShapeDtypeStruct((1, BATCH), jnp.float32),
        compiler_params=pltpu.CompilerParams(
            dimension_semantics=("parallel",)),
        interpret=interpret,
    )(umlp, imlp, umf, imf, W1uT, W1iT, b1c, W2T, b2c, wl1c, wl2c, bl)


def kernel(user_indices, item_indices, U_mf, I_mf, U_mlp, I_mlp,
           W1, b1, W2, b2, Wl, bl):
    uidx = user_indices.astype(jnp.int32)
    iidx = item_indices.astype(jnp.int32)
    g_umf, g_imf, g_umlp, g_imlp = _sc_gather(
        uidx, iidx, U_mf.T, I_mf.T, U_mlp.T, I_mlp.T)
    W1uT = W1[:D].T
    W1iT = W1[D:].T
    b1c = b1.reshape(D, 1)
    b2c = b2.reshape(16, 1)
    wl1c = Wl[:16, 0].reshape(16, 1)
    wl2c = Wl[16:, 0].reshape(D, 1)
    blr = bl.reshape(1, 1)
    out = _tc_dense(g_umlp, g_imlp, g_umf, g_imf,
                    W1uT, W1iT, b1c, W2.T, b2c, wl1c, wl2c, blr)
    return out.reshape(BATCH, 1)
